# baseline (speedup 1.0000x reference)
import functools

import jax
import jax.numpy as jnp
from jax import lax
from jax.experimental import pallas as pl
from jax.experimental.pallas import tpu as pltpu
from jax.experimental.pallas import tpu_sc as plsc

NB = 16384
ND = 32
NE = 1000000
NW = 32
EH_TOTAL = 7813
EH_BASE = EH_TOTAL // NW
EH_REM = EH_TOTAL % NW
CH_EH = 8
CHUNK_E = CH_EH * 128
NCHUNK = (EH_BASE + 1 + CH_EH - 1) // CH_EH
NCHUNK_PAD = NCHUNK + (NCHUNK % 2)
L1CAP = 1024
CHCAP = 128
HF_LEN = NB * ND + 4096
DUMP = NB * ND

BPW = NB // NW


def _k1_body(ent_hbm, h_hbm, t_hbm, hf_hbm, tf_hbm,
             strip, lst_e, lst_p, ch_e, ch_p, buf,
             stag00, stag01, stag10, stag11, sidx00, sidx01, sidx10, sidx11,
             sem_a, sem_b, sem_s0, sem_s1, sem_c):
    stags = ((stag00, stag01), (stag10, stag11))
    sidxs = ((sidx00, sidx01), (sidx10, sidx11))
    w = lax.axis_index("s") * 2 + lax.axis_index("c")
    ehlo = w * EH_BASE + jnp.minimum(w, EH_REM)
    ehhi = ehlo + EH_BASE + (w < EH_REM).astype(jnp.int32)
    iota = lax.iota(jnp.int32, 16)
    zero16 = jnp.zeros((16,), jnp.int32)

    e_lo = ehlo * 128
    e_hi = ehhi * 128
    runs = []
    for s, idx_hbm in ((0, h_hbm), (1, t_hbm)):
        run = zero16
        for blk in range(NB // CHUNK_E):
            pltpu.sync_copy(idx_hbm.at[pl.ds(blk * CHUNK_E, CHUNK_E)], strip)

            def l1(v, run, blk=blk, s=s):
                e = strip[pl.ds(v * 16, 16)]
                m = (e >= e_lo) & (e < e_hi)
                cs = plsc.cumsum(m.astype(jnp.int32))
                slot = jnp.minimum(run + cs - 1, L1CAP - 1) + s * L1CAP
                plsc.store_scatter(lst_e, [slot], e, mask=m)
                pos = blk * CHUNK_E + v * 16 + iota
                plsc.store_scatter(lst_p, [slot], pos, mask=m)
                return run + plsc.all_reduce_population_count(m)

            run = lax.fori_loop(0, CHUNK_E // 16, l1, run)
        runs.append(lax.reduce_max(run, (0,)))

    nvr_h = (runs[0] + 15) >> 4
    nvr_t = (runs[1] + 15) >> 4

    dh0 = iota >> 3
    dl0 = iota & 7
    dh1 = dh0 + 2

    def init_stage(v, carry):
        for s in range(2):
            for sl in range(2):
                stags[s][sl][pl.ds(v * 16, 16)] = jnp.zeros((16,), jnp.float32)
                sidxs[s][sl][pl.ds(v * 16, 16)] = DUMP + iota
        return carry

    lax.fori_loop(0, CHCAP * 2, init_stage, 0)

    def init_ch(v, carry):
        ch_e[pl.ds(v * 16, 16)] = zero16
        ch_p[pl.ds(v * 16, 16)] = zero16
        return carry

    lax.fori_loop(0, CHCAP // 16, init_ch, 0)

    def fire_chunk(c, nb):
        c0 = ehlo + CH_EH * c
        dma_eh = jnp.minimum(c0, EH_TOTAL - CH_EH)
        sem = sem_a if nb == 0 else sem_b
        for dh in range(4):
            pltpu.async_copy(
                ent_hbm.at[pl.ds(dh * 8, 8), pl.ds(dma_eh * 128, CHUNK_E)],
                buf.at[nb, dh], sem)

    def drain_chunk(nb):
        sem = sem_a if nb == 0 else sem_b
        for dh in range(4):
            pltpu.make_async_copy(
                ent_hbm.at[pl.ds(dh * 8, 8), pl.ds(0, CHUNK_E)],
                buf.at[nb, dh], sem).wait()

    fire_chunk(0, 0)
    fire_chunk(1, 1)
    for s, (out_hbm, sem_s) in enumerate(((hf_hbm, sem_s0), (tf_hbm, sem_s1))):
        for sl in range(2):
            pltpu.async_copy(stags[s][sl], out_hbm.at[sidxs[s][sl]], sem_s)

    def chunk_step(g, carry):
        for b in range(2):
            cc = g * 2 + b
            drain_chunk(b)
            for s, (out_hbm, sem_s) in enumerate(
                    ((hf_hbm, sem_s0), (tf_hbm, sem_s1))):
                pltpu.make_async_copy(
                    stags[s][b], out_hbm.at[sidxs[s][b]], sem_s).wait()

            c0 = ehlo + CH_EH * cc
            dma_e0 = jnp.minimum(c0, EH_TOTAL - CH_EH) * 128
            ce_lo = c0 * 128
            ce_hi = jnp.minimum(c0 + CH_EH, ehhi) * 128
            for s, (nvr, out_hbm, sem_s) in enumerate(
                    ((nvr_h, hf_hbm, sem_s0), (nvr_t, tf_hbm, sem_s1))):
                base = s * L1CAP

                def l2(v, crun, base=base, ce_lo=ce_lo, ce_hi=ce_hi,
                       dma_e0=dma_e0):
                    e = lst_e[pl.ds(base + v * 16, 16)]
                    p = lst_p[pl.ds(base + v * 16, 16)]
                    m = (e >= ce_lo) & (e < ce_hi)
                    cs = plsc.cumsum(m.astype(jnp.int32))
                    slot = jnp.minimum(crun + cs - 1, CHCAP - 1)
                    plsc.store_scatter(ch_e, [slot], e - dma_e0, mask=m)
                    plsc.store_scatter(ch_p, [slot], p, mask=m)
                    return crun + plsc.all_reduce_population_count(m)

                crun = lax.fori_loop(0, nvr, l2, zero16)
                n_c = lax.reduce_max(crun, (0,))

                def extract(v, carry2, s=s, b=b, n_c=n_c):
                    ev = ch_e[pl.ds(v * 16, 16)]
                    pv = ch_p[pl.ds(v * 16, 16)]
                    for j in range(16):
                        e_loc = ev[j]
                        pos = pv[j]
                        live = v * 16 + j < n_c
                        eb = jnp.full((16,), e_loc, jnp.int32)
                        g0 = plsc.load_gather(buf.at[b], [dh0, dl0, eb])
                        g1 = plsc.load_gather(buf.at[b], [dh1, dl0, eb])
                        slot = v * 16 + j
                        stags[s][b][pl.ds(slot * 32, 16)] = g0
                        stags[s][b][pl.ds(slot * 32 + 16, 16)] = g1
                        sidxs[s][b][pl.ds(slot * 32, 16)] = jnp.where(
                            live, pos * 32 + iota, DUMP + iota)
                        sidxs[s][b][pl.ds(slot * 32 + 16, 16)] = jnp.where(
                            live, pos * 32 + 16 + iota, DUMP + iota)
                    return carry2

                lax.fori_loop(0, (n_c + 15) >> 4, extract, 0)
                pltpu.async_copy(stags[s][b], out_hbm.at[sidxs[s][b]], sem_s)
            fire_chunk(cc + 2, b)
        return carry

    lax.fori_loop(0, NCHUNK_PAD // 2, chunk_step, 0)

    for b in range(2):
        drain_chunk(b)
        for s, (out_hbm, sem_s) in enumerate(
                ((hf_hbm, sem_s0), (tf_hbm, sem_s1))):
            pltpu.make_async_copy(
                stags[s][b], out_hbm.at[sidxs[s][b]], sem_s).wait()


def _k2_body(hf_hbm, tf_hbm, rel_hbm, r_hbm, out_hbm,
             h_v, t_v, r_v, ri_v, out_v, sem):
    w = lax.axis_index("s") * 2 + lax.axis_index("c")
    base = w * BPW
    pltpu.sync_copy(r_hbm.at[pl.ds(base, BPW)], ri_v)
    cps = [
        pltpu.async_copy(hf_hbm.at[pl.ds(base * ND, BPW * ND)], h_v, sem),
        pltpu.async_copy(tf_hbm.at[pl.ds(base * ND, BPW * ND)], t_v, sem),
        pltpu.async_copy(rel_hbm.at[ri_v], r_v, sem),
    ]
    for cp in cps:
        cp.wait()

    lane = lax.iota(jnp.int32, 16)

    def body(blk, carry):
        vec = jnp.zeros((16,), jnp.float32)
        for i in range(16):
            row = blk * 16 + i
            h0 = h_v[pl.ds(row * 32, 16)]
            h1 = h_v[pl.ds(row * 32 + 16, 16)]
            t0 = t_v[pl.ds(row * 32, 16)]
            t1 = t_v[pl.ds(row * 32 + 16, 16)]
            r0 = r_v[row, pl.ds(0, 16)]
            r1 = r_v[row, pl.ds(16, 16)]
            acc = h0 * r0 * t0 + h1 * r1 * t1
            vec = jnp.where(lane == i, jnp.sum(acc), vec)
        out_v[pl.ds(blk * 16, 16)] = vec
        return carry

    lax.fori_loop(0, BPW // 16, body, 0)
    pltpu.sync_copy(out_v, out_hbm.at[pl.ds(base, BPW)])


def kernel(entity, relation, h_index, t_index, r_index):
    mesh = plsc.VectorSubcoreMesh(core_axis_name="c", subcore_axis_name="s")
    ent_t = entity.T

    k1 = functools.partial(
        pl.kernel,
        mesh=mesh,
        out_type=(
            jax.ShapeDtypeStruct((HF_LEN,), jnp.float32),
            jax.ShapeDtypeStruct((HF_LEN,), jnp.float32),
        ),
        compiler_params=pltpu.CompilerParams(
            needs_layout_passes=False, use_tc_tiling_on_sc=True),
        scratch_types=[
            pltpu.VMEM((CHUNK_E,), jnp.int32),
            pltpu.VMEM((2 * L1CAP,), jnp.int32),
            pltpu.VMEM((2 * L1CAP,), jnp.int32),
            pltpu.VMEM((CHCAP,), jnp.int32),
            pltpu.VMEM((CHCAP,), jnp.int32),
            pltpu.VMEM((2, 4, 8, CHUNK_E), jnp.float32),
            pltpu.VMEM((CHCAP * 32,), jnp.float32),
            pltpu.VMEM((CHCAP * 32,), jnp.float32),
            pltpu.VMEM((CHCAP * 32,), jnp.float32),
            pltpu.VMEM((CHCAP * 32,), jnp.float32),
            pltpu.VMEM((CHCAP * 32,), jnp.int32),
            pltpu.VMEM((CHCAP * 32,), jnp.int32),
            pltpu.VMEM((CHCAP * 32,), jnp.int32),
            pltpu.VMEM((CHCAP * 32,), jnp.int32),
            pltpu.SemaphoreType.DMA,
            pltpu.SemaphoreType.DMA,
            pltpu.SemaphoreType.DMA,
            pltpu.SemaphoreType.DMA,
            pltpu.SemaphoreType.DMA,
        ],
    )(_k1_body)
    hf, tf = k1(ent_t, h_index.astype(jnp.int32), t_index.astype(jnp.int32))

    k2 = functools.partial(
        pl.kernel,
        mesh=mesh,
        out_type=jax.ShapeDtypeStruct((NB,), jnp.float32),
        compiler_params=pltpu.CompilerParams(
            needs_layout_passes=False, use_tc_tiling_on_sc=False),
        scratch_types=[
            pltpu.VMEM((BPW * ND,), jnp.float32),
            pltpu.VMEM((BPW * ND,), jnp.float32),
            pltpu.VMEM((BPW, ND), jnp.float32),
            pltpu.VMEM((BPW,), jnp.int32),
            pltpu.VMEM((BPW,), jnp.float32),
            pltpu.SemaphoreType.DMA,
        ],
    )(_k2_body)
    return k2(hf, tf, relation, r_index.astype(jnp.int32))
